# baseline (speedup 1.0000x reference)
import jax
import jax.numpy as jnp
from jax.experimental import pallas as pl
from jax.experimental.pallas import tpu as pltpu

B, T, C, N = 4, 12, 4, 512
K = 5
ROW_BLK = 256
STRIP = 8


def _fused_kernel(w_ref, g_ref, o_ref):
    w = jax.nn.softmax(w_ref[0, 0, :])
    neg = jnp.float32(-jnp.inf)

    def strip_body(i, carry):
        r0 = i * STRIP
        g = g_ref[0, :, pl.ds(r0, STRIP), :]
        s = jnp.maximum(jnp.sum(jnp.abs(g), axis=-1, keepdims=True), 1e-12)
        coef = w[:, None, None] / s
        agg = jnp.sum(g * coef, axis=0)

        thr = jnp.max(agg, axis=-1, keepdims=True)
        for _ in range(K - 1):
            below = jnp.where(agg < thr, agg, neg)
            thr = jnp.max(below, axis=-1, keepdims=True)

        masked = jnp.where(agg >= thr, agg, 0.0)
        denom = jnp.maximum(jnp.sum(masked, axis=-1, keepdims=True), 1e-12)
        o_ref[0, pl.ds(r0, STRIP), :] = masked / denom
        return carry

    jax.lax.fori_loop(0, ROW_BLK // STRIP, strip_body, 0)


@jax.jit
def kernel(graphs, weight):
    g = graphs.reshape(B * T, C, N, N)
    w = jnp.broadcast_to(weight.reshape(1, T, C), (B, T, C)).reshape(B * T, 1, C)

    grid = (B * T, N // ROW_BLK)
    out = pl.pallas_call(
        _fused_kernel,
        grid=grid,
        in_specs=[
            pl.BlockSpec((1, 1, C), lambda bt, ib: (bt, 0, 0)),
            pl.BlockSpec((1, C, ROW_BLK, N), lambda bt, ib: (bt, 0, ib, 0)),
        ],
        out_specs=pl.BlockSpec((1, ROW_BLK, N), lambda bt, ib: (bt, ib, 0)),
        out_shape=jax.ShapeDtypeStruct((B * T, N, N), jnp.float32),
        compiler_params=pltpu.CompilerParams(
            dimension_semantics=("parallel", "parallel"),
        ),
    )(w, g)
    return out.reshape(B, T, N, N)


# speedup vs baseline: 3.9171x; 3.9171x over previous
import jax
import jax.numpy as jnp
from jax import lax
from jax.experimental import pallas as pl
from jax.experimental.pallas import tpu as pltpu
from jax.experimental.pallas import tpu_sc as plsc

B, T, C, N = 4, 12, 4, 512
K = 5
L = 16
NW = 32
RPW = N // NW
NCHUNK = N // L

NEG = -jnp.inf


_GDN = lax.GatherDimensionNumbers(
    offset_dims=(), collapsed_slice_dims=(0,), start_index_map=(0,))


def _shuf(v, lane, sh):
    idx = jnp.bitwise_xor(lane, sh)
    return lax.gather(
        v, idx[:, None], _GDN, slice_sizes=(1,),
        mode=lax.GatherScatterMode.PROMISE_IN_BOUNDS)


def _allmax(v, lane):
    for sh in (8, 4, 2, 1):
        v = jnp.maximum(v, _shuf(v, lane, sh))
    return v


def _allsum(v, lane):
    for sh in (8, 4, 2, 1):
        v = v + _shuf(v, lane, sh)
    return v


def _sc_kernel(g_hbm, w_hbm, out_hbm, g_v, w_v, agg_v, out_v):
    wid = lax.axis_index("s") * 2 + lax.axis_index("c")
    r0 = wid * RPW
    lane = lax.iota(jnp.int32, L)

    def slice_body(bt, _):
        pltpu.sync_copy(g_hbm.at[bt, :, pl.ds(r0, RPW), :], g_v)
        pltpu.sync_copy(w_hbm.at[bt], w_v)

        wvec = w_v[...]
        mx = _allmax(wvec, lane)
        evec = jnp.exp(wvec - mx)
        ew = [_allmax(jnp.where(lane == c, evec, NEG), lane) for c in range(C)]

        def row_body(r, _):
            invs = []
            for c in range(C):
                acc = jnp.abs(g_v[c, r, pl.ds(0, L)])
                for j in range(1, NCHUNK):
                    acc = acc + jnp.abs(g_v[c, r, pl.ds(j * L, L)])
                s = jnp.maximum(_allsum(acc, lane), 1e-12)
                invs.append(ew[c] / s)

            top = [jnp.full((L,), NEG, jnp.float32) for _ in range(K)]
            for j in range(NCHUNK):
                a = g_v[0, r, pl.ds(j * L, L)] * invs[0]
                for c in range(1, C):
                    a = a + g_v[c, r, pl.ds(j * L, L)] * invs[c]
                agg_v[r, pl.ds(j * L, L)] = a
                v = a
                for t in range(K):
                    hi = jnp.maximum(top[t], v)
                    v = jnp.minimum(top[t], v)
                    top[t] = hi

            thr = jnp.full((L,), jnp.inf, jnp.float32)
            for _ in range(K):
                m = jnp.full((L,), NEG, jnp.float32)
                for t in range(K):
                    m = jnp.maximum(m, jnp.where(top[t] < thr, top[t], NEG))
                thr = _allmax(m, lane)

            den = jnp.zeros((L,), jnp.float32)
            for j in range(NCHUNK):
                a = agg_v[r, pl.ds(j * L, L)]
                masked = jnp.where(a >= thr, a, 0.0)
                out_v[r, pl.ds(j * L, L)] = masked
                den = den + masked
            invd = 1.0 / jnp.maximum(_allsum(den, lane), 1e-12)
            for j in range(NCHUNK):
                out_v[r, pl.ds(j * L, L)] = out_v[r, pl.ds(j * L, L)] * invd
            return 0

        lax.fori_loop(0, RPW, row_body, 0)
        pltpu.sync_copy(out_v, out_hbm.at[bt, pl.ds(r0, RPW), :])
        return 0

    lax.fori_loop(0, B * T, slice_body, 0)


@jax.jit
def kernel(graphs, weight):
    g = graphs.reshape(B * T, C, N, N)
    w16 = jnp.full((T, L), -1e30, jnp.float32)
    w16 = w16.at[:, :C].set(weight.reshape(T, C))
    w16 = jnp.broadcast_to(w16.reshape(1, T, L), (B, T, L)).reshape(B * T, L)

    mesh = plsc.VectorSubcoreMesh(core_axis_name="c", subcore_axis_name="s")
    run = pl.kernel(
        _sc_kernel,
        mesh=mesh,
        out_type=jax.ShapeDtypeStruct((B * T, N, N), jnp.float32),
        scratch_types=[
            pltpu.VMEM((C, RPW, N), jnp.float32),
            pltpu.VMEM((L,), jnp.float32),
            pltpu.VMEM((RPW, N), jnp.float32),
            pltpu.VMEM((RPW, N), jnp.float32),
        ],
    )
    out = run(g, w16)
    return out.reshape(B, T, N, N)


# speedup vs baseline: 16.5412x; 4.2228x over previous
import jax
import jax.numpy as jnp
from jax.experimental import pallas as pl
from jax.experimental.pallas import tpu as pltpu

B, T, C, N = 4, 12, 4, 512
K = 5
ROW_BLK = 512
STRIP = 32


def _fused_kernel(w_ref, g_ref, o_ref):
    w = jax.nn.softmax(w_ref[0, 0, :])
    neg = jnp.float32(-jnp.inf)

    for i in range(ROW_BLK // STRIP):
        r0 = i * STRIP
        g = g_ref[0, :, r0:r0 + STRIP, :]
        s = jnp.maximum(jnp.sum(jnp.abs(g), axis=-1, keepdims=True), 1e-12)
        coef = w[:, None, None] / s
        agg = jnp.sum(g * coef, axis=0)

        thr = jnp.max(agg, axis=-1, keepdims=True)
        for _ in range(K - 1):
            below = jnp.where(agg < thr, agg, neg)
            thr = jnp.max(below, axis=-1, keepdims=True)

        masked = jnp.where(agg >= thr, agg, 0.0)
        denom = jnp.maximum(jnp.sum(masked, axis=-1, keepdims=True), 1e-12)
        o_ref[0, r0:r0 + STRIP, :] = masked / denom


@jax.jit
def kernel(graphs, weight):
    g = graphs.reshape(B * T, C, N, N)
    w = jnp.broadcast_to(weight.reshape(1, T, C), (B, T, C)).reshape(B * T, 1, C)

    grid = (B * T, N // ROW_BLK)
    out = pl.pallas_call(
        _fused_kernel,
        grid=grid,
        in_specs=[
            pl.BlockSpec((1, 1, C), lambda bt, ib: (bt, 0, 0)),
            pl.BlockSpec((1, C, ROW_BLK, N), lambda bt, ib: (bt, 0, ib, 0)),
        ],
        out_specs=pl.BlockSpec((1, ROW_BLK, N), lambda bt, ib: (bt, ib, 0)),
        out_shape=jax.ShapeDtypeStruct((B * T, N, N), jnp.float32),
        compiler_params=pltpu.CompilerParams(
            dimension_semantics=("parallel", "parallel"),
        ),
    )(w, g)
    return out.reshape(B, T, N, N)
